# baseline (speedup 1.0000x reference)
import functools

import numpy as np
import jax
import jax.numpy as jnp
from jax.experimental import pallas as pl
from jax.experimental.pallas import tpu as pltpu

_EPS = 0.1
_TILE = 32


@functools.lru_cache(maxsize=2)
def _sampling_consts(B, A):
    with jax.ensure_compile_time_eval():
        key = jax.random.key(42)
        ku, ks, km = jax.random.split(key, 3)
        rand_sample = jax.random.categorical(ku, jnp.zeros((B, A), jnp.float32))
        gumbel = jax.random.gumbel(ks, (B, A), jnp.float32)
        mask = (jax.random.uniform(km, (B,)) <= _EPS).astype(jnp.int32)
    return np.concatenate(
        [np.asarray(gumbel, np.float32),
         np.asarray(rand_sample, np.float32)[:, None],
         np.asarray(mask, np.float32)[:, None]], axis=1)


def _body(tid_ref, chg_ref, x_ref, *refs):
    pols = refs[:_TILE]
    c_ref, out_ref, pstack = refs[_TILE:]
    A = c_ref.shape[-1] - 2
    NS = chg_ref.shape[0] // _TILE
    i = pl.program_id(0)
    for j in range(_TILE):
        @pl.when(chg_ref[NS * j + i] != 0)
        def _(j=j):
            pstack[:, A * j:A * (j + 1)] = pols[j][0]
    X = x_ref[0]
    Y = jnp.dot(X, pstack[:, :],
                preferred_element_type=jnp.float32)
    eye = jax.lax.broadcasted_iota(jnp.int32, (_TILE, _TILE, 1), 0) == \
        jax.lax.broadcasted_iota(jnp.int32, (_TILE, _TILE, 1), 1)
    probs = jnp.sum(jnp.where(eye, Y.reshape(_TILE, _TILE, A), 0.0), axis=1)
    consts = c_ref[0]
    logits = jnp.log(jnp.clip(probs, 1e-30, None)) + consts[:, :A]
    samp = jnp.argmax(logits, axis=1).astype(jnp.int32)[:, None]
    rs = consts[:, A:A + 1].astype(jnp.int32)
    out_ref[0] = jnp.where(consts[:, A + 1:A + 2] == 1.0, rs, samp)


def kernel(x, task_id, policy):
    B, K = x.shape
    T, _, A = policy.shape
    NS = B // _TILE

    cpack = _sampling_consts(B, A)

    order = jnp.argsort(task_id)
    tid_s = task_id[order]
    order2 = order.reshape(_TILE, NS).T.reshape(B)
    tid2 = tid_s.reshape(_TILE, NS)
    chg = jnp.concatenate(
        [jnp.ones((_TILE, 1), jnp.int32),
         (tid2[:, 1:] != tid2[:, :-1]).astype(jnp.int32)], axis=1).reshape(B)

    pol_spec = [
        pl.BlockSpec((1, K, A),
                     (lambda i, tid, chg, j=j: (tid[NS * j + i], 0, 0)))
        for j in range(_TILE)
    ]
    grid_spec = pltpu.PrefetchScalarGridSpec(
        num_scalar_prefetch=2,
        grid=(NS,),
        in_specs=[pl.BlockSpec((1, _TILE, K), lambda i, tid, chg: (i, 0, 0))]
        + pol_spec
        + [pl.BlockSpec((1, _TILE, A + 2), lambda i, tid, chg: (i, 0, 0))],
        out_specs=pl.BlockSpec((1, _TILE, 1), lambda i, tid, chg: (i, 0, 0)),
        scratch_shapes=[pltpu.VMEM((K, _TILE * A), jnp.float32)],
    )
    a = pl.pallas_call(
        _body,
        grid_spec=grid_spec,
        out_shape=jax.ShapeDtypeStruct((NS, _TILE, 1), jnp.int32),
        compiler_params=pltpu.CompilerParams(
            dimension_semantics=("arbitrary",)),
    )(tid_s, chg, x[order2].reshape(NS, _TILE, K),
      *([policy] * _TILE),
      jnp.asarray(cpack)[order2].reshape(NS, _TILE, A + 2))
    return jnp.zeros((B,), jnp.int32).at[order2].set(a.reshape(B))


# speedup vs baseline: 1.1495x; 1.1495x over previous
import functools

import numpy as np
import jax
import jax.numpy as jnp
from jax.experimental import pallas as pl
from jax.experimental.pallas import tpu as pltpu

_EPS = 0.1
_TILE = 64


@functools.lru_cache(maxsize=2)
def _sampling_consts(B, A):
    with jax.ensure_compile_time_eval():
        key = jax.random.key(42)
        ku, ks, km = jax.random.split(key, 3)
        rand_sample = jax.random.categorical(ku, jnp.zeros((B, A), jnp.float32))
        gumbel = jax.random.gumbel(ks, (B, A), jnp.float32)
        mask = (jax.random.uniform(km, (B,)) <= _EPS).astype(jnp.int32)
    return (np.asarray(rand_sample, dtype=np.int32),
            np.asarray(gumbel, dtype=np.float32),
            np.asarray(mask, dtype=np.int32))


def _body(tid_ref, x_ref, *refs):
    pols = refs[:_TILE]
    gum_ref, rs_ref, mk_ref, out_ref = refs[_TILE:]
    A = gum_ref.shape[-1]
    X = x_ref[0]
    P = jnp.concatenate([p[0] for p in pols], axis=1)
    Y = jnp.dot(X, P, preferred_element_type=jnp.float32)
    probs = jnp.concatenate(
        [Y[j:j + 1, A * j:A * (j + 1)] for j in range(_TILE)], axis=0)
    logits = jnp.log(jnp.clip(probs, 1e-30, None)) + gum_ref[0]
    samp = jnp.argmax(logits, axis=1).astype(jnp.int32)[:, None]
    out_ref[0] = jnp.where(mk_ref[0] == 1, rs_ref[0], samp)


def kernel(x, task_id, policy):
    B, K = x.shape
    T, _, A = policy.shape
    NS = B // _TILE

    rand_sample, gumbel, mask = _sampling_consts(B, A)

    order = jnp.argsort(task_id)
    tid_s = task_id[order]
    order2 = order.reshape(_TILE, NS).T.reshape(B)

    pol_spec = [
        pl.BlockSpec((1, K, A),
                     (lambda i, tid, j=j: (tid[NS * j + i], 0, 0)))
        for j in range(_TILE)
    ]
    grid_spec = pltpu.PrefetchScalarGridSpec(
        num_scalar_prefetch=1,
        grid=(NS,),
        in_specs=[pl.BlockSpec((1, _TILE, K), lambda i, tid: (i, 0, 0))]
        + pol_spec
        + [pl.BlockSpec((1, _TILE, A), lambda i, tid: (i, 0, 0)),
           pl.BlockSpec((1, _TILE, 1), lambda i, tid: (i, 0, 0)),
           pl.BlockSpec((1, _TILE, 1), lambda i, tid: (i, 0, 0))],
        out_specs=pl.BlockSpec((1, _TILE, 1), lambda i, tid: (i, 0, 0)),
    )
    a = pl.pallas_call(
        _body,
        grid_spec=grid_spec,
        out_shape=jax.ShapeDtypeStruct((NS, _TILE, 1), jnp.int32),
        compiler_params=pltpu.CompilerParams(
            dimension_semantics=("arbitrary",)),
    )(tid_s, x[order2].reshape(NS, _TILE, K),
      *([policy] * _TILE),
      jnp.asarray(gumbel)[order2].reshape(NS, _TILE, A),
      jnp.asarray(rand_sample)[order2].reshape(NS, _TILE, 1),
      jnp.asarray(mask)[order2].reshape(NS, _TILE, 1))
    return jnp.zeros((B,), jnp.int32).at[order2].set(a.reshape(B))


# speedup vs baseline: 1.1633x; 1.0120x over previous
import functools

import numpy as np
import jax
import jax.numpy as jnp
from jax.experimental import pallas as pl
from jax.experimental.pallas import tpu as pltpu

_EPS = 0.1
_TILE = 64


@functools.lru_cache(maxsize=2)
def _sampling_consts(B, A):
    with jax.ensure_compile_time_eval():
        key = jax.random.key(42)
        ku, ks, km = jax.random.split(key, 3)
        rand_sample = jax.random.categorical(ku, jnp.zeros((B, A), jnp.float32))
        gumbel = jax.random.gumbel(ks, (B, A), jnp.float32)
        mask = (jax.random.uniform(km, (B,)) <= _EPS).astype(jnp.int32)
    return np.concatenate(
        [np.asarray(gumbel, np.float32),
         np.asarray(rand_sample, np.float32)[:, None],
         np.asarray(mask, np.float32)[:, None]], axis=1)


def _body(tid_ref, x_ref, *refs):
    pols = refs[:_TILE]
    c_ref, out_ref = refs[_TILE:]
    A = c_ref.shape[-1] - 2
    X = x_ref[0]
    P = jnp.concatenate([p[0] for p in pols], axis=1)
    Y = jnp.dot(X, P, preferred_element_type=jnp.float32)
    probs = jnp.concatenate(
        [Y[j:j + 1, A * j:A * (j + 1)] for j in range(_TILE)], axis=0)
    consts = c_ref[0]
    logits = jnp.log(jnp.clip(probs, 1e-30, None)) + consts[:, :A]
    samp = jnp.argmax(logits, axis=1).astype(jnp.int32)[:, None]
    rs = consts[:, A:A + 1].astype(jnp.int32)
    out_ref[0] = jnp.where(consts[:, A + 1:A + 2] == 1.0, rs, samp)


def kernel(x, task_id, policy):
    B, K = x.shape
    T, _, A = policy.shape
    NS = B // _TILE

    cpack = _sampling_consts(B, A)

    order = jnp.argsort(task_id)
    tid_s = task_id[order]
    order2 = order.reshape(_TILE, NS).T.reshape(B)

    pol_spec = [
        pl.BlockSpec((1, K, A),
                     (lambda i, tid, j=j: (tid[NS * j + i], 0, 0)))
        for j in range(_TILE)
    ]
    grid_spec = pltpu.PrefetchScalarGridSpec(
        num_scalar_prefetch=1,
        grid=(NS,),
        in_specs=[pl.BlockSpec((1, _TILE, K), lambda i, tid: (i, 0, 0))]
        + pol_spec
        + [pl.BlockSpec((1, _TILE, A + 2), lambda i, tid: (i, 0, 0))],
        out_specs=pl.BlockSpec((1, _TILE, 1), lambda i, tid: (i, 0, 0)),
    )
    a = pl.pallas_call(
        _body,
        grid_spec=grid_spec,
        out_shape=jax.ShapeDtypeStruct((NS, _TILE, 1), jnp.int32),
        compiler_params=pltpu.CompilerParams(
            dimension_semantics=("arbitrary",)),
    )(tid_s, x[order2].reshape(NS, _TILE, K),
      *([policy] * _TILE),
      jnp.asarray(cpack)[order2].reshape(NS, _TILE, A + 2))
    return jnp.zeros((B,), jnp.int32).at[order2].set(a.reshape(B))
